# Initial kernel scaffold; baseline (speedup 1.0000x reference)
import jax
import jax.numpy as jnp
from jax.experimental import pallas as pl


def kernel(input_ids, kg_enc_input, cross_attn, last_hidden_state, entity_emb, rel_emb, W_mlp, b_mlp, W_lin, W_li, Wq, Wk, Wv, Wo, W_out, Wg, bg, Wc, bc):
    raise NotImplementedError("write your pallas kernel here")


# speedup vs baseline: 1.9725x; 1.9725x over previous
import functools

import jax
import jax.numpy as jnp
from jax.experimental import pallas as pl
from jax.experimental.pallas import tpu as pltpu

_B, _MAX_LEN, _SRC_LEN = 8, 64, 128
_N1, _N2 = 50, 10
_NT = _N1 * _N2
_NTP = 512
_VOCAB = 50000
_T_EMBED, _HIDDEN, _HEADS = 300, 768, 8
_DK = _HIDDEN // _HEADS

_VT = 2048
_NVT = (_VOCAB + _VT - 1) // _VT


def _wg_eff_kernel(w_ref, wg_ref, o_ref):
    t = pl.program_id(0)

    @pl.when(t == 0)
    def _():
        o_ref[...] = jnp.zeros_like(o_ref)

    valid = (t * _VT + jax.lax.broadcasted_iota(jnp.int32, (1, _VT), 1)) < _VOCAB
    w = jnp.where(valid, w_ref[...], 0.0)
    wg = jnp.where(valid, wg_ref[...], 0.0)
    o_ref[...] += jnp.sum(w * wg, axis=1, keepdims=True)


def _main_kernel(outh_ref, wout_ref, dlg_ref, ids_ref, attn_ref, tail_ref,
                 o_ref):
    t = pl.program_id(0)
    vbase = t * _VT

    outh = outh_ref[0].astype(jnp.bfloat16)
    w = wout_ref[...].astype(jnp.bfloat16)
    acc = jnp.dot(outh, w, preferred_element_type=jnp.float32)

    ids = ids_ref[0].reshape(_SRC_LEN, 1)
    vid_c = vbase + jax.lax.broadcasted_iota(jnp.int32, (_SRC_LEN, _VT), 1)
    mc = (ids == vid_c).astype(jnp.bfloat16)
    acc += jnp.dot(dlg_ref[0].astype(jnp.bfloat16), mc,
                   preferred_element_type=jnp.float32)

    tail = tail_ref[0].reshape(_NTP, 1)
    vid_k = vbase + jax.lax.broadcasted_iota(jnp.int32, (_NTP, _VT), 1)
    mk = (tail == vid_k).astype(jnp.bfloat16)
    acc += jnp.dot(attn_ref[0].astype(jnp.bfloat16), mk,
                   preferred_element_type=jnp.float32)

    o_ref[0] = acc


def kernel(input_ids, kg_enc_input, cross_attn, last_hidden_state, entity_emb,
           rel_emb, W_mlp, b_mlp, W_lin, W_li, Wq, Wk, Wv, Wo, W_out, Wg, bg,
           Wc, bc):
    B, M, S, NT = _B, _MAX_LEN, _SRC_LEN, _NT

    head = kg_enc_input[..., 0].reshape(B, NT)
    rel = kg_enc_input[..., 1].reshape(B, NT)
    tail = kg_enc_input[..., 2].reshape(B, NT)
    head_e = jnp.take(entity_emb, head, axis=0)
    rel_e = jnp.take(rel_emb, rel, axis=0)
    tail_e = jnp.take(entity_emb, tail, axis=0)
    triple = jnp.concatenate([head_e, rel_e, tail_e], axis=-1)
    triple = triple @ W_mlp + b_mlp
    triple = triple @ W_lin

    dlg_attn = jnp.mean(cross_attn, axis=1)
    out_h = last_hidden_state @ W_li

    q = (out_h @ Wq).reshape(B, M, _HEADS, _DK).transpose(0, 2, 1, 3)
    k = (triple @ Wk).reshape(B, NT, _HEADS, _DK).transpose(0, 2, 1, 3)
    v = (triple @ Wv).reshape(B, NT, _HEADS, _DK).transpose(0, 2, 1, 3)
    scores = (q @ k.transpose(0, 1, 3, 2)) / jnp.sqrt(jnp.float32(_DK))
    p = jax.nn.softmax(scores, axis=-1)
    ctx = (p @ v).transpose(0, 2, 1, 3).reshape(B, M, _HIDDEN)
    mid = ctx @ Wo
    attn = jnp.mean(p, axis=1)

    wg_eff = pl.pallas_call(
        _wg_eff_kernel,
        grid=(_NVT,),
        in_specs=[
            pl.BlockSpec((_HIDDEN, _VT), lambda t: (0, t)),
            pl.BlockSpec((1, _VT), lambda t: (0, t)),
        ],
        out_specs=pl.BlockSpec((_HIDDEN, 1), lambda t: (0, 0)),
        out_shape=jax.ShapeDtypeStruct((_HIDDEN, 1), jnp.float32),
    )(W_out, Wg.reshape(1, _VOCAB))

    p_gen = jax.nn.sigmoid(out_h @ wg_eff + bg)
    p_con = jax.nn.sigmoid(mid @ Wc + bc)

    outh_s = (1.0 - p_con) * p_gen * out_h
    dlg_s = (1.0 - p_con) * (1.0 - p_gen) * dlg_attn
    attn_s = p_con * attn
    attn_s = jnp.pad(attn_s, ((0, 0), (0, 0), (0, _NTP - NT)))
    tail_p = jnp.pad(tail, ((0, 0), (0, _NTP - NT))).reshape(B, 1, _NTP)
    ids3 = input_ids.reshape(B, 1, S)

    out = pl.pallas_call(
        _main_kernel,
        grid=(_NVT, B),
        in_specs=[
            pl.BlockSpec((1, M, _HIDDEN), lambda t, b: (b, 0, 0)),
            pl.BlockSpec((_HIDDEN, _VT), lambda t, b: (0, t)),
            pl.BlockSpec((1, M, S), lambda t, b: (b, 0, 0)),
            pl.BlockSpec((1, 1, S), lambda t, b: (b, 0, 0)),
            pl.BlockSpec((1, M, _NTP), lambda t, b: (b, 0, 0)),
            pl.BlockSpec((1, 1, _NTP), lambda t, b: (b, 0, 0)),
        ],
        out_specs=pl.BlockSpec((1, M, _VT), lambda t, b: (b, 0, t)),
        out_shape=jax.ShapeDtypeStruct((B, M, _VOCAB), jnp.float32),
        compiler_params=pltpu.CompilerParams(
            dimension_semantics=("arbitrary", "arbitrary"),
        ),
    )(outh_s, W_out, dlg_s, ids3, attn_s, tail_p)
    return out


# speedup vs baseline: 2.4890x; 1.2619x over previous
import functools

import jax
import jax.numpy as jnp
from jax.experimental import pallas as pl
from jax.experimental.pallas import tpu as pltpu

_B, _MAX_LEN, _SRC_LEN = 8, 64, 128
_N1, _N2 = 50, 10
_NT = _N1 * _N2
_NTP = 512
_VOCAB = 50000
_T_EMBED, _HIDDEN, _HEADS = 300, 768, 8
_DK = _HIDDEN // _HEADS

_VT = 2048
_NVT = (_VOCAB + _VT - 1) // _VT


def _wg_eff_kernel(w_ref, wg_ref, o_ref):
    t = pl.program_id(0)

    @pl.when(t == 0)
    def _():
        o_ref[...] = jnp.zeros_like(o_ref)

    valid = (t * _VT + jax.lax.broadcasted_iota(jnp.int32, (1, _VT), 1)) < _VOCAB
    w = jnp.where(valid, w_ref[...], 0.0)
    wg = jnp.where(valid, wg_ref[...], 0.0)
    o_ref[...] += jnp.sum(w * wg, axis=1, keepdims=True)


_NSC = _SRC_LEN + _NTP


def _main_kernel(outh_ref, wout_ref, sv_ref, idx_ref, o_ref):
    t = pl.program_id(0)

    w = wout_ref[...].astype(jnp.bfloat16)
    acc = jnp.dot(outh_ref[...].astype(jnp.bfloat16), w,
                  preferred_element_type=jnp.float32)

    vid = t * _VT + jax.lax.broadcasted_iota(jnp.int32, (_NSC, _VT), 1)
    rows = []
    for b in range(_B):
        idxb = idx_ref[b].reshape(_NSC, 1)
        m = (idxb == vid).astype(jnp.bfloat16)
        rows.append(jnp.dot(sv_ref[b].astype(jnp.bfloat16), m,
                            preferred_element_type=jnp.float32))
    o_ref[...] = acc + jnp.concatenate(rows, axis=0)


def kernel(input_ids, kg_enc_input, cross_attn, last_hidden_state, entity_emb,
           rel_emb, W_mlp, b_mlp, W_lin, W_li, Wq, Wk, Wv, Wo, W_out, Wg, bg,
           Wc, bc):
    B, M, S, NT = _B, _MAX_LEN, _SRC_LEN, _NT

    head = kg_enc_input[..., 0].reshape(B, NT)
    rel = kg_enc_input[..., 1].reshape(B, NT)
    tail = kg_enc_input[..., 2].reshape(B, NT)
    head_e = jnp.take(entity_emb, head, axis=0)
    rel_e = jnp.take(rel_emb, rel, axis=0)
    tail_e = jnp.take(entity_emb, tail, axis=0)
    triple = jnp.concatenate([head_e, rel_e, tail_e], axis=-1)
    triple = triple @ W_mlp + b_mlp
    triple = triple @ W_lin

    dlg_attn = jnp.mean(cross_attn, axis=1)
    out_h = last_hidden_state @ W_li

    q = (out_h @ Wq).reshape(B, M, _HEADS, _DK).transpose(0, 2, 1, 3)
    k = (triple @ Wk).reshape(B, NT, _HEADS, _DK).transpose(0, 2, 1, 3)
    v = (triple @ Wv).reshape(B, NT, _HEADS, _DK).transpose(0, 2, 1, 3)
    scores = (q @ k.transpose(0, 1, 3, 2)) / jnp.sqrt(jnp.float32(_DK))
    p = jax.nn.softmax(scores, axis=-1)
    ctx = (p @ v).transpose(0, 2, 1, 3).reshape(B, M, _HIDDEN)
    mid = ctx @ Wo
    attn = jnp.mean(p, axis=1)

    wg_eff = pl.pallas_call(
        _wg_eff_kernel,
        grid=(_NVT,),
        in_specs=[
            pl.BlockSpec((_HIDDEN, _VT), lambda t: (0, t)),
            pl.BlockSpec((1, _VT), lambda t: (0, t)),
        ],
        out_specs=pl.BlockSpec((_HIDDEN, 1), lambda t: (0, 0)),
        out_shape=jax.ShapeDtypeStruct((_HIDDEN, 1), jnp.float32),
    )(W_out, Wg.reshape(1, _VOCAB))

    p_gen = jax.nn.sigmoid(out_h @ wg_eff + bg)
    p_con = jax.nn.sigmoid(mid @ Wc + bc)

    outh_s = ((1.0 - p_con) * p_gen * out_h).reshape(B * M, _HIDDEN)
    dlg_s = (1.0 - p_con) * (1.0 - p_gen) * dlg_attn
    attn_s = p_con * attn
    attn_s = jnp.pad(attn_s, ((0, 0), (0, 0), (0, _NTP - NT)))
    sv = jnp.concatenate([dlg_s, attn_s], axis=2)
    tail_p = jnp.pad(tail, ((0, 0), (0, _NTP - NT)))
    idx = jnp.concatenate([input_ids, tail_p], axis=1).reshape(B, 1, _NSC)

    out = pl.pallas_call(
        _main_kernel,
        grid=(_NVT,),
        in_specs=[
            pl.BlockSpec((B * M, _HIDDEN), lambda t: (0, 0)),
            pl.BlockSpec((_HIDDEN, _VT), lambda t: (0, t)),
            pl.BlockSpec((B, M, _NSC), lambda t: (0, 0, 0)),
            pl.BlockSpec((B, 1, _NSC), lambda t: (0, 0, 0)),
        ],
        out_specs=pl.BlockSpec((B * M, _VT), lambda t: (0, t)),
        out_shape=jax.ShapeDtypeStruct((B * M, _VOCAB), jnp.float32),
        compiler_params=pltpu.CompilerParams(
            dimension_semantics=("arbitrary",),
        ),
    )(outh_s, W_out, sv, idx)
    return out.reshape(B, M, _VOCAB)
